# baseline (speedup 1.0000x reference)
import dataclasses
import functools

import jax
import jax.numpy as jnp
from jax import lax
from jax.experimental import pallas as pl
from jax.experimental.pallas import tpu as pltpu
from jax.experimental.pallas import tpu_sc as plsc

CAP = 1_000_000
NEG = -3.0e38
IBIG = 2**31 - 1
PADCAP = ((CAP + 127) // 128) * 128

LEN = 31488
SC_CHUNK = LEN // 2
LAST_SC_BASE = PADCAP - LEN

_sc_mesh = plsc.VectorSubcoreMesh(core_axis_name="c", subcore_axis_name="s")
_sc_cp = pltpu.CompilerParams()
if "needs_layout_passes" in pltpu.CompilerParams.__dataclass_fields__:
    _sc_cp = dataclasses.replace(_sc_cp, needs_layout_passes=False)


def _sc_trunc(v):
    b = plsc.bitcast(v, jnp.int32)
    r = (b + 0x7FFF + ((b >> 16) & 1)) & ~0xFFFF
    return plsc.bitcast(r, jnp.float32)


@functools.partial(
    pl.kernel, mesh=_sc_mesh, compiler_params=_sc_cp,
    out_type=[jax.ShapeDtypeStruct((32, 16), jnp.float32),
              jax.ShapeDtypeStruct((32, 16), jnp.int32)],
    scratch_types=[
        pltpu.VMEM((4, SC_CHUNK), jnp.float32),
        pltpu.VMEM((4, SC_CHUNK), jnp.float32),
        pltpu.VMEM((4, 16), jnp.float32),
        pltpu.VMEM((16,), jnp.int32),
        pltpu.VMEM((16,), jnp.float32),
        pltpu.VMEM((16,), jnp.int32),
        pltpu.SemaphoreType.DMA,
        pltpu.SemaphoreType.DMA,
    ],
)
def _sc_kernel(bufT_hbm, phb_hbm, kib_hbm, omax_hbm, oidx_hbm,
               chunk_a, chunk_b, ph_v, kill_v, vmax_v, vidx_v, sem_a, sem_b):
    c = lax.axis_index("c")
    s = lax.axis_index("s")
    wid = c * 16 + s
    base = pl.multiple_of(jnp.minimum(wid * LEN, LAST_SC_BASE), 128)
    cp_a = pltpu.make_async_copy(
        bufT_hbm.at[pl.ds(0, 4), pl.ds(base, SC_CHUNK)], chunk_a, sem_a)
    cp_b = pltpu.make_async_copy(
        bufT_hbm.at[pl.ds(0, 4), pl.ds(base + SC_CHUNK, SC_CHUNK)], chunk_b, sem_b)
    cp_a.start()
    cp_b.start()
    pltpu.sync_copy(phb_hbm, ph_v)
    pltpu.sync_copy(kib_hbm, kill_v)
    p0 = ph_v[0, :]
    p1 = ph_v[1, :]
    p2 = ph_v[2, :]
    p3 = ph_v[3, :]
    kill = kill_v[...]
    iota = lax.iota(jnp.int32, 16)

    def _scan(chunk_v, cbase, carry):
        def _step(g, mv):
            vmax, vidx = mv
            t0 = _sc_trunc(chunk_v[0, pl.ds(g, 16)])
            t1 = _sc_trunc(chunk_v[1, pl.ds(g, 16)])
            t2 = _sc_trunc(chunk_v[2, pl.ds(g, 16)])
            t3 = _sc_trunc(chunk_v[3, pl.ds(g, 16)])
            sim = (t0 * p0 + t1 * p1) + (t2 * p2 + t3 * p3)
            gidx = (cbase + g) + iota
            upd = (sim > vmax) & (gidx != kill) & (gidx < CAP)
            return (jnp.where(upd, sim, vmax), jnp.where(upd, gidx, vidx))

        def body(i, mv):
            g = i * 64
            mv = _step(g, mv)
            mv = _step(g + 16, mv)
            mv = _step(g + 32, mv)
            mv = _step(g + 48, mv)
            return mv

        return lax.fori_loop(0, SC_CHUNK // 64, body, carry)

    carry = (jnp.full((16,), NEG, jnp.float32), jnp.full((16,), IBIG, jnp.int32))
    cp_a.wait()
    carry = _scan(chunk_a, base, carry)
    cp_b.wait()
    vmax, vidx = _scan(chunk_b, base + SC_CHUNK, carry)

    vmax_v[...] = vmax
    vidx_v[...] = vidx
    pltpu.sync_copy(vmax_v, omax_hbm.at[wid])
    pltpu.sync_copy(vidx_v, oidx_hbm.at[wid])


def _merge_body(idx_ref, phs_ref, scmax_ref, scidx_ref,
                trajT_ref, bufT_any, out_ref, gi_sm, wrow_ref, sem):
    idx = idx_ref[0]

    scm = scmax_ref[...]
    gmax = jnp.max(scm)
    gi0 = jnp.min(jnp.where(scm == gmax, scidx_ref[...], IBIG))
    gi_sm[0] = gi0
    gi = gi_sm[0]

    j0 = pl.multiple_of((gi // 128) * 128, 128)
    cp = pltpu.make_async_copy(bufT_any.at[:, pl.ds(j0, 128)], wrow_ref, sem)
    cp.start()
    cp.wait()
    colw = jax.lax.broadcasted_iota(jnp.int32, (8, 128), 1)
    w = jnp.where(colw == gi - j0, wrow_ref[...], 0.0)
    roww = jnp.sum(w, axis=1, keepdims=True)
    row_act = roww[4:7, :]

    asum = jnp.sum(trajT_ref[...], axis=1, keepdims=True)
    theta = jnp.sqrt(jnp.sum(asum * asum))
    axis = asum / (theta + 1e-8)
    qr = jnp.cos(theta)
    qi = axis * jnp.sin(theta)
    to_f = lambda x: x.astype(jnp.bfloat16).astype(jnp.float32)
    sim_e = (to_f(qr) * phs_ref[0] + to_f(qi[0, 0]) * phs_ref[1]
             + to_f(qi[1, 0]) * phs_ref[2] + to_f(qi[2, 0]) * phs_ref[3])
    win_e = (sim_e > gmax) | ((sim_e == gmax) & (idx < gi))

    res = jnp.where(win_e, asum, row_act)
    out_ref[...] = jnp.broadcast_to(res, (3, 128))


def _merge_call(idx, phase, scmax, scidx, trajT, bufT):
    return pl.pallas_call(
        _merge_body,
        grid=(1,),
        in_specs=[
            pl.BlockSpec(memory_space=pltpu.SMEM),
            pl.BlockSpec(memory_space=pltpu.SMEM),
            pl.BlockSpec((32, 16), lambda i: (0, 0)),
            pl.BlockSpec((32, 16), lambda i: (0, 0)),
            pl.BlockSpec((3, 8192), lambda i: (0, 0)),
            pl.BlockSpec(memory_space=pl.ANY),
        ],
        out_specs=pl.BlockSpec((3, 128), lambda i: (0, 0)),
        out_shape=jax.ShapeDtypeStruct((3, 128), jnp.float32),
        scratch_shapes=[
            pltpu.SMEM((1,), jnp.int32),
            pltpu.VMEM((8, 128), jnp.float32),
            pltpu.SemaphoreType.DMA,
        ],
    )(idx, phase, scmax, scidx, trajT, bufT)


def kernel(trajectory_lie_elements, value, current_phase, buffer, ptr):
    del value
    idx = (jnp.asarray(ptr, jnp.int32) % CAP).reshape(1)
    bufT = buffer.T
    trajT = trajectory_lie_elements.T
    phb = jnp.broadcast_to(current_phase.reshape(4, 1), (4, 16))
    kib = jnp.broadcast_to(idx, (16,))

    sc_max, sc_idx = _sc_kernel(bufT, phb, kib)
    out = _merge_call(idx, current_phase, sc_max, sc_idx, trajT, bufT)
    return out[:, 0]


# speedup vs baseline: 1.2048x; 1.2048x over previous
import dataclasses
import functools

import jax
import jax.numpy as jnp
from jax import lax
from jax.experimental import pallas as pl
from jax.experimental.pallas import tpu as pltpu
from jax.experimental.pallas import tpu_sc as plsc

CAP = 1_000_000
NEG = -3.0e38
IBIG = 2**31 - 1
PADCAP = ((CAP + 127) // 128) * 128

LEN = 31488
SC_CHUNK = LEN // 2
LAST_SC_BASE = PADCAP - LEN

_sc_mesh = plsc.VectorSubcoreMesh(core_axis_name="c", subcore_axis_name="s")
_sc_cp = pltpu.CompilerParams()
if "needs_layout_passes" in pltpu.CompilerParams.__dataclass_fields__:
    _sc_cp = dataclasses.replace(_sc_cp, needs_layout_passes=False)


def _sc_trunc(v):
    return plsc.unpack(plsc.pack(v, v, format=plsc.PackFormat.INTERLEAVED),
                       format=plsc.PackFormat.INTERLEAVED)[0]


@functools.partial(
    pl.kernel, mesh=_sc_mesh, compiler_params=_sc_cp,
    out_type=[jax.ShapeDtypeStruct((32, 16), jnp.float32),
              jax.ShapeDtypeStruct((32, 16), jnp.int32)],
    scratch_types=[
        pltpu.VMEM((4, SC_CHUNK), jnp.float32),
        pltpu.VMEM((4, SC_CHUNK), jnp.float32),
        pltpu.VMEM((4, 16), jnp.float32),
        pltpu.VMEM((16,), jnp.int32),
        pltpu.VMEM((16,), jnp.float32),
        pltpu.VMEM((16,), jnp.int32),
        pltpu.SemaphoreType.DMA,
        pltpu.SemaphoreType.DMA,
    ],
)
def _sc_kernel(bufT_hbm, phb_hbm, kib_hbm, omax_hbm, oidx_hbm,
               chunk_a, chunk_b, ph_v, kill_v, vmax_v, vidx_v, sem_a, sem_b):
    c = lax.axis_index("c")
    s = lax.axis_index("s")
    wid = c * 16 + s
    base = pl.multiple_of(jnp.minimum(wid * LEN, LAST_SC_BASE), 128)
    cp_a = pltpu.make_async_copy(
        bufT_hbm.at[pl.ds(0, 4), pl.ds(base, SC_CHUNK)], chunk_a, sem_a)
    cp_b = pltpu.make_async_copy(
        bufT_hbm.at[pl.ds(0, 4), pl.ds(base + SC_CHUNK, SC_CHUNK)], chunk_b, sem_b)
    cp_a.start()
    cp_b.start()
    pltpu.sync_copy(phb_hbm, ph_v)
    pltpu.sync_copy(kib_hbm, kill_v)
    p0 = ph_v[0, :]
    p1 = ph_v[1, :]
    p2 = ph_v[2, :]
    p3 = ph_v[3, :]
    kill = kill_v[...]
    iota = lax.iota(jnp.int32, 16)

    def _scan(chunk_v, cbase, carry):
        def _step(g, mv):
            vmax, vidx = mv
            t0 = _sc_trunc(chunk_v[0, pl.ds(g, 16)])
            t1 = _sc_trunc(chunk_v[1, pl.ds(g, 16)])
            t2 = _sc_trunc(chunk_v[2, pl.ds(g, 16)])
            t3 = _sc_trunc(chunk_v[3, pl.ds(g, 16)])
            sim = (t0 * p0 + t1 * p1) + (t2 * p2 + t3 * p3)
            gidx = (cbase + g) + iota
            upd = (sim > vmax) & (gidx != kill) & (gidx < CAP)
            return (jnp.where(upd, sim, vmax), jnp.where(upd, gidx, vidx))

        def body(i, mv):
            g = i * 64
            mv = _step(g, mv)
            mv = _step(g + 16, mv)
            mv = _step(g + 32, mv)
            mv = _step(g + 48, mv)
            return mv

        return lax.fori_loop(0, SC_CHUNK // 64, body, carry)

    carry = (jnp.full((16,), NEG, jnp.float32), jnp.full((16,), IBIG, jnp.int32))
    cp_a.wait()
    carry = _scan(chunk_a, base, carry)
    cp_b.wait()
    vmax, vidx = _scan(chunk_b, base + SC_CHUNK, carry)

    vmax_v[...] = vmax
    vidx_v[...] = vidx
    pltpu.sync_copy(vmax_v, omax_hbm.at[wid])
    pltpu.sync_copy(vidx_v, oidx_hbm.at[wid])


def _merge_body(idx_ref, phs_ref, scmax_ref, scidx_ref,
                trajT_ref, bufT_any, out_ref, gi_sm, wrow_ref, sem):
    idx = idx_ref[0]

    scm = scmax_ref[...]
    gmax = jnp.max(scm)
    gi0 = jnp.min(jnp.where(scm == gmax, scidx_ref[...], IBIG))
    gi_sm[0] = gi0
    gi = gi_sm[0]

    j0 = pl.multiple_of((gi // 128) * 128, 128)
    cp = pltpu.make_async_copy(bufT_any.at[:, pl.ds(j0, 128)], wrow_ref, sem)
    cp.start()
    cp.wait()
    colw = jax.lax.broadcasted_iota(jnp.int32, (8, 128), 1)
    w = jnp.where(colw == gi - j0, wrow_ref[...], 0.0)
    roww = jnp.sum(w, axis=1, keepdims=True)
    row_act = roww[4:7, :]

    asum = jnp.sum(trajT_ref[...], axis=1, keepdims=True)
    theta = jnp.sqrt(jnp.sum(asum * asum))
    axis = asum / (theta + 1e-8)
    qr = jnp.cos(theta)
    qi = axis * jnp.sin(theta)
    to_f = lambda x: x.astype(jnp.bfloat16).astype(jnp.float32)
    sim_e = (to_f(qr) * phs_ref[0] + to_f(qi[0, 0]) * phs_ref[1]
             + to_f(qi[1, 0]) * phs_ref[2] + to_f(qi[2, 0]) * phs_ref[3])
    win_e = (sim_e > gmax) | ((sim_e == gmax) & (idx < gi))

    res = jnp.where(win_e, asum, row_act)
    out_ref[...] = jnp.broadcast_to(res, (3, 128))


def _merge_call(idx, phase, scmax, scidx, trajT, bufT):
    return pl.pallas_call(
        _merge_body,
        grid=(1,),
        in_specs=[
            pl.BlockSpec(memory_space=pltpu.SMEM),
            pl.BlockSpec(memory_space=pltpu.SMEM),
            pl.BlockSpec((32, 16), lambda i: (0, 0)),
            pl.BlockSpec((32, 16), lambda i: (0, 0)),
            pl.BlockSpec((3, 8192), lambda i: (0, 0)),
            pl.BlockSpec(memory_space=pl.ANY),
        ],
        out_specs=pl.BlockSpec((3, 128), lambda i: (0, 0)),
        out_shape=jax.ShapeDtypeStruct((3, 128), jnp.float32),
        scratch_shapes=[
            pltpu.SMEM((1,), jnp.int32),
            pltpu.VMEM((8, 128), jnp.float32),
            pltpu.SemaphoreType.DMA,
        ],
    )(idx, phase, scmax, scidx, trajT, bufT)


def kernel(trajectory_lie_elements, value, current_phase, buffer, ptr):
    del value
    idx = (jnp.asarray(ptr, jnp.int32) % CAP).reshape(1)
    bufT = buffer.T
    trajT = trajectory_lie_elements.T
    phb = jnp.broadcast_to(current_phase.reshape(4, 1), (4, 16))
    kib = jnp.broadcast_to(idx, (16,))

    sc_max, sc_idx = _sc_kernel(bufT, phb, kib)
    out = _merge_call(idx, current_phase, sc_max, sc_idx, trajT, bufT)
    return out[:, 0]
